# baseline (speedup 1.0000x reference)
import functools

import jax
import jax.numpy as jnp
from jax import lax
from jax.experimental import pallas as pl
from jax.experimental.pallas import tpu as pltpu
from jax.experimental.pallas import tpu_sc as plsc

_N = 10000
_E = 320000
_D = 128
_DG = 16
_NC = 2
_NS = 16
_NW = _NC * _NS
_NP = 10240
_EP = 327680
_EPW = _EP // _NW
_B = 64
_NCH = _EPW // _B
_RPT = _NP // _NS

_mesh = plsc.VectorSubcoreMesh(core_axis_name="c", subcore_axis_name="s")


@functools.partial(
    pl.kernel,
    out_type=(
        jax.ShapeDtypeStruct((_NC, _NP, _D), jnp.float32),
        jax.ShapeDtypeStruct((_NC, _NP, _DG), jnp.float32),
    ),
    mesh=_mesh,
    scratch_types=[
        pltpu.VMEM_SHARED((_NP, _D), jnp.float32),
        pltpu.VMEM_SHARED((_NP, _DG), jnp.float32),
        pltpu.VMEM((_NCH, _B), jnp.int32),
        pltpu.VMEM((_NCH, _B), jnp.int32),
        pltpu.VMEM((_B, _D), jnp.float32),
        pltpu.VMEM((_B, _D), jnp.float32),
        pltpu.VMEM((_B, _DG), jnp.float32),
        pltpu.VMEM((32, _DG), jnp.float32),
        pltpu.SemaphoreType.DMA,
        pltpu.SemaphoreType.DMA,
    ],
    compiler_params=pltpu.CompilerParams(use_tc_tiling_on_sc=False),
)
def _sc_aggregate(x, src, dst, out_f, out_d, acc, dacc, src_v, dst_v,
                  rows0, rows1, ones_v, zd, gsem0, gsem1):
    c = lax.axis_index("c")
    s = lax.axis_index("s")
    wid = s * _NC + c
    off = pl.multiple_of(s * _RPT, 8)

    zero16 = jnp.zeros((16,), jnp.float32)

    def zrows_row(r, carry):
        for k in range(_D // 16):
            rows0[r, pl.ds(16 * k, 16)] = zero16
        return carry

    lax.fori_loop(0, _B, zrows_row, 0)

    def small_row(r, carry):
        ones_v[r, pl.ds(0, 16)] = jnp.full((16,), 1.0 / _DG, jnp.float32)
        return carry

    lax.fori_loop(0, _B, small_row, 0)

    def zd_row(r, carry):
        zd[r, pl.ds(0, 16)] = zero16
        return carry

    lax.fori_loop(0, 32, zd_row, 0)

    def zero_blk(z, carry):
        zoff = pl.multiple_of(off + z * _B, 8)
        pltpu.sync_copy(rows0, acc.at[pl.ds(zoff, _B)])
        return carry

    lax.fori_loop(0, _RPT // _B, zero_blk, 0)

    def zero_dblk(z, carry):
        zoff = pl.multiple_of(off + z * 32, 8)
        pltpu.sync_copy(zd, dacc.at[pl.ds(zoff, 32)])
        return carry

    lax.fori_loop(0, _RPT // 32, zero_dblk, 0)

    pltpu.sync_copy(src.at[wid], src_v)
    pltpu.sync_copy(dst.at[wid], dst_v)
    plsc.subcore_barrier()

    pltpu.async_copy(x.at[src_v.at[0]], rows0, gsem0)

    def body(i, carry):
        j0 = 2 * i
        pltpu.make_async_copy(x.at[src_v.at[j0]], rows0, gsem0).wait()
        pltpu.async_copy(x.at[src_v.at[j0 + 1]], rows1, gsem1)
        pltpu.sync_copy(rows0, acc.at[dst_v.at[j0]], add=True)
        pltpu.sync_copy(ones_v, dacc.at[dst_v.at[j0]], add=True)
        pltpu.make_async_copy(x.at[src_v.at[j0 + 1]], rows1, gsem1).wait()

        @pl.when(i < _NCH // 2 - 1)
        def _():
            pltpu.async_copy(x.at[src_v.at[j0 + 2]], rows0, gsem0)

        pltpu.sync_copy(rows1, acc.at[dst_v.at[j0 + 1]], add=True)
        pltpu.sync_copy(ones_v, dacc.at[dst_v.at[j0 + 1]], add=True)
        return carry

    lax.fori_loop(0, _NCH // 2, body, 0)
    plsc.subcore_barrier()
    pltpu.sync_copy(acc.at[pl.ds(off, _RPT)], out_f.at[c, pl.ds(off, _RPT)])
    pltpu.sync_copy(dacc.at[pl.ds(off, _RPT)], out_d.at[c, pl.ds(off, _RPT)])


_RB = 1000


def _tc_body(x_ref, p0_ref, p1_ref, d0_ref, d1_ref, ws_ref, wn_ref, bias_ref, o_ref):
    deg = jnp.sum(d0_ref[...] + d1_ref[...], axis=1, keepdims=True)
    h = (p0_ref[...] + p1_ref[...]) * (1.0 / jnp.maximum(deg, 1.0))
    o_ref[...] = (
        jnp.dot(x_ref[...], ws_ref[...], preferred_element_type=jnp.float32)
        + jnp.dot(h, wn_ref[...], preferred_element_type=jnp.float32)
        + bias_ref[...]
    )


_tc_dense = pl.pallas_call(
    _tc_body,
    grid=(_N // _RB,),
    in_specs=[
        pl.BlockSpec((_RB, _D), lambda i: (i, 0)),
        pl.BlockSpec((_RB, _D), lambda i: (i, 0)),
        pl.BlockSpec((_RB, _D), lambda i: (i, 0)),
        pl.BlockSpec((_RB, _DG), lambda i: (i, 0)),
        pl.BlockSpec((_RB, _DG), lambda i: (i, 0)),
        pl.BlockSpec((_D, _D), lambda i: (0, 0)),
        pl.BlockSpec((_D, _D), lambda i: (0, 0)),
        pl.BlockSpec((1, _D), lambda i: (0, 0)),
    ],
    out_specs=pl.BlockSpec((_RB, _D), lambda i: (i, 0)),
    out_shape=jax.ShapeDtypeStruct((_N, _D), jnp.float32),
)


def kernel(x, edge_index, W_self, b_self, W_neigh, b_neigh):
    ei = edge_index.astype(jnp.int32)
    pad_src = jnp.zeros((_EP - _E,), jnp.int32)
    pad_dst = jnp.full((_EP - _E,), _NP - 1, jnp.int32)
    src = jnp.concatenate([ei[0], pad_src]).reshape(_NW, _NCH, _B)
    dst = jnp.concatenate([ei[1], pad_dst]).reshape(_NW, _NCH, _B)
    pf, pd = _sc_aggregate(x, src, dst)
    bias = (b_self + b_neigh)[None, :]
    return _tc_dense(x, pf[0], pf[1], pd[0], pd[1], W_self.T, W_neigh.T, bias)


# speedup vs baseline: 2.0175x; 2.0175x over previous
import functools

import jax
import jax.numpy as jnp
from jax import lax
from jax.experimental import pallas as pl
from jax.experimental.pallas import tpu as pltpu
from jax.experimental.pallas import tpu_sc as plsc

_N = 10000
_E = 320000
_D = 128
_DP = 144
_NC = 2
_NS = 16
_NW = _NC * _NS
_EPW = _E // _NW
_B = 100
_NCH = _EPW // _B
_NPAIR = _NCH // 2
_NP = 10112
_RPT = _NP // _NS

_mesh = plsc.VectorSubcoreMesh(core_axis_name="c", subcore_axis_name="s")


@functools.partial(
    pl.kernel,
    out_type=jax.ShapeDtypeStruct((_NC, _NP, _DP), jnp.float32),
    mesh=_mesh,
    scratch_types=[
        pltpu.VMEM_SHARED((_NP, _DP), jnp.float32),
        pltpu.VMEM((_NCH, _B), jnp.int32),
        pltpu.VMEM((4, _B), jnp.int32),
        pltpu.VMEM((_B, _DP), jnp.float32),
        pltpu.VMEM((_B, _DP), jnp.float32),
        pltpu.SemaphoreType.DMA,
        pltpu.SemaphoreType.DMA,
        pltpu.SemaphoreType.DMA,
        pltpu.SemaphoreType.DMA,
        pltpu.SemaphoreType.DMA,
    ],
    compiler_params=pltpu.CompilerParams(use_tc_tiling_on_sc=False),
)
def _sc_aggregate(xa, srcq, dst, out, acc, dst_v, srcb, rows0, rows1,
                  gsem0, gsem1, ssem0, ssem1, isem):
    c = lax.axis_index("c")
    s = lax.axis_index("s")
    wid = s * _NC + c
    off = pl.multiple_of(s * _RPT, 8)

    zero16 = jnp.zeros((16,), jnp.float32)

    def zrows_row(r, carry):
        for k in range(_DP // 16):
            rows0[r, pl.ds(16 * k, 16)] = zero16
        return carry

    lax.fori_loop(0, _B, zrows_row, 0)

    def zero_blk(z, carry):
        zoff = pl.multiple_of(off + z * 96, 8)
        pltpu.sync_copy(rows0.at[pl.ds(0, 96)], acc.at[pl.ds(zoff, 96)])
        return carry

    lax.fori_loop(0, 6, zero_blk, 0)
    tail = pl.multiple_of(off + 576, 8)
    pltpu.sync_copy(rows0.at[pl.ds(0, 56)], acc.at[pl.ds(tail, 56)])

    pltpu.sync_copy(dst.at[wid], dst_v)
    plsc.subcore_barrier()

    pltpu.async_copy(srcq.at[wid, 0], srcb.at[pl.ds(0, 2)], isem).wait()
    pltpu.async_copy(xa.at[srcb.at[0]], rows0, gsem0)
    pltpu.async_copy(xa.at[srcb.at[1]], rows1, gsem1)
    pltpu.async_copy(srcq.at[wid, 1], srcb.at[pl.ds(2, 2)], isem)

    def body(i, carry):
        j0 = 2 * i
        p = lax.rem(i, 2)
        pn = 1 - p
        pltpu.make_async_copy(xa.at[srcb.at[2 * p]], rows0, gsem0).wait()
        pltpu.async_copy(rows0, acc.at[dst_v.at[j0]], ssem0, add=True)
        pltpu.make_async_copy(xa.at[srcb.at[2 * p + 1]], rows1, gsem1).wait()
        pltpu.async_copy(rows1, acc.at[dst_v.at[j0 + 1]], ssem1, add=True)

        pltpu.make_async_copy(rows0, acc.at[dst_v.at[j0]], ssem0).wait()

        @pl.when(i < _NPAIR - 1)
        def _():
            pltpu.make_async_copy(
                srcq.at[wid, i + 1], srcb.at[pl.ds(2 * pn, 2)], isem).wait()
            pltpu.async_copy(xa.at[srcb.at[2 * pn]], rows0, gsem0)

        pltpu.make_async_copy(rows1, acc.at[dst_v.at[j0 + 1]], ssem1).wait()

        @pl.when(i < _NPAIR - 1)
        def _():
            pltpu.async_copy(xa.at[srcb.at[2 * pn + 1]], rows1, gsem1)

        @pl.when(i < _NPAIR - 2)
        def _():
            pltpu.async_copy(srcq.at[wid, i + 2], srcb.at[pl.ds(2 * p, 2)], isem)

        return carry

    lax.fori_loop(0, _NPAIR, body, 0)
    plsc.subcore_barrier()
    pltpu.sync_copy(acc.at[pl.ds(off, _RPT)], out.at[c, pl.ds(off, _RPT)])


_RB = 1000


def _tc_body(x_ref, p0_ref, p1_ref, ws_ref, wn_ref, bias_ref, o_ref):
    p = p0_ref[...] + p1_ref[...]
    deg = jnp.sum(p[:, _D:], axis=1, keepdims=True)
    h = p[:, :_D] * (1.0 / jnp.maximum(deg, 1.0))
    o_ref[...] = (
        jnp.dot(x_ref[...], ws_ref[...], preferred_element_type=jnp.float32)
        + jnp.dot(h, wn_ref[...], preferred_element_type=jnp.float32)
        + bias_ref[...]
    )


_tc_dense = pl.pallas_call(
    _tc_body,
    grid=(_N // _RB,),
    in_specs=[
        pl.BlockSpec((_RB, _D), lambda i: (i, 0)),
        pl.BlockSpec((_RB, _DP), lambda i: (i, 0)),
        pl.BlockSpec((_RB, _DP), lambda i: (i, 0)),
        pl.BlockSpec((_D, _D), lambda i: (0, 0)),
        pl.BlockSpec((_D, _D), lambda i: (0, 0)),
        pl.BlockSpec((1, _D), lambda i: (0, 0)),
    ],
    out_specs=pl.BlockSpec((_RB, _D), lambda i: (i, 0)),
    out_shape=jax.ShapeDtypeStruct((_N, _D), jnp.float32),
)


def kernel(x, edge_index, W_self, b_self, W_neigh, b_neigh):
    ei = edge_index.astype(jnp.int32)
    srcq = ei[0].reshape(_NW, _NPAIR, 2, _B)
    dst = ei[1].reshape(_NW, _NCH, _B)
    xa = jnp.concatenate(
        [x, jnp.ones((_N, 1), jnp.float32), jnp.zeros((_N, _DP - _D - 1), jnp.float32)],
        axis=1,
    )
    partials = _sc_aggregate(xa, srcq, dst)
    bias = (b_self + b_neigh)[None, :]
    return _tc_dense(x, partials[0], partials[1], W_self.T, W_neigh.T, bias)
